# baseline (speedup 1.0000x reference)
import functools

import jax
import jax.numpy as jnp
from jax import lax
from jax.experimental import pallas as pl
from jax.experimental.pallas import tpu as pltpu
from jax.experimental.pallas import tpu_sc as plsc

NN = 10000
EE = 320000
HH = 145
LL = 4
CC = 21
GG = 512
H2 = 72

NC, NS, LANE = 2, 16, 16
NWK = NC * NS
R = 320
NP = NWK * R
HP = 160
CP = 32
HHP = 128
CAP = 16384
WE = 2000
NWIN = EE // WE
K = 128
NVEC = HP // LANE

def _wid():
    return lax.axis_index("s") * NC + lax.axis_index("c")


@functools.cache
def _sc_kernels():
    mesh = plsc.VectorSubcoreMesh(
        core_axis_name="c", subcore_axis_name="s",
        num_cores=NC, num_subcores=NS)
    cparams = pltpu.CompilerParams(needs_layout_passes=False,
                                   use_tc_tiling_on_sc=False)
    prep = pl.kernel(
        _prep_body,
        compiler_params=cparams,
        out_type=[
            jax.ShapeDtypeStruct((NWK, CAP), jnp.int32),
            jax.ShapeDtypeStruct((NWK, CAP), jnp.int32),
            jax.ShapeDtypeStruct((NWK, 16), jnp.int32),
            jax.ShapeDtypeStruct((NP,), jnp.int32),
        ],
        mesh=mesh,
        scratch_types=[
            pltpu.VMEM((WE,), jnp.int32),
            pltpu.VMEM((WE,), jnp.int32),
            pltpu.VMEM((WE,), jnp.int32),
            pltpu.VMEM((WE,), jnp.int32),
            pltpu.VMEM((CAP,), jnp.int32),
            pltpu.VMEM((CAP,), jnp.int32),
            pltpu.VMEM(((R + 1) * 16,), jnp.int32),
            pltpu.VMEM((R,), jnp.int32),
            pltpu.VMEM((16,), jnp.int32),
            pltpu.SemaphoreType.DMA,
            pltpu.SemaphoreType.DMA,
            pltpu.SemaphoreType.DMA,
            pltpu.SemaphoreType.DMA,
        ],
    )
    conv = pl.kernel(
        _conv_body,
        compiler_params=cparams,
        out_type=jax.ShapeDtypeStruct((NP * HP,), jnp.float32),
        mesh=mesh,
        scratch_types=[
            pltpu.VMEM(((R + 1) * HP,), jnp.float32),
            pltpu.VMEM((K,), jnp.int32),
            pltpu.VMEM((K,), jnp.int32),
            pltpu.VMEM((K, HP), jnp.float32),
            pltpu.VMEM((K, HP), jnp.float32),
            pltpu.VMEM((K,), jnp.int32),
            pltpu.VMEM((K,), jnp.int32),
            pltpu.VMEM((16,), jnp.int32),
            pltpu.SemaphoreType.DMA,
            pltpu.SemaphoreType.DMA,
            pltpu.SemaphoreType.DMA,
            pltpu.SemaphoreType.DMA,
            pltpu.SemaphoreType.DMA,
            pltpu.SemaphoreType.DMA,
        ],
    )
    return prep, conv


def _prep_body(erow, ecol, rows_o, cols_o, cnts_o, deg_o,
               colw0, roww0, colw1, roww1, lrows, lcols,
               degacc, degcomp, cntv, cs0, rs0, cs1, rs1):
    wid = _wid()
    lo = wid * R

    def issue(slot_col, slot_row, csem, rsem, w):
        pltpu.async_copy(ecol.at[pl.ds(w * WE, WE)], slot_col, csem)
        pltpu.async_copy(erow.at[pl.ds(w * WE, WE)], slot_row, rsem)

    def wait(slot_col, slot_row, csem, rsem):
        pltpu.make_async_copy(ecol.at[pl.ds(0, WE)], slot_col, csem).wait()
        pltpu.make_async_copy(erow.at[pl.ds(0, WE)], slot_row, rsem).wait()

    issue(colw0, roww0, cs0, rs0, 0)
    issue(colw1, roww1, cs1, rs1, 1)

    def process(colw, roww, cur):
        def group(g, cur):
            col = colw[pl.ds(g * LANE, LANE)]
            row = roww[pl.ds(g * LANE, LANE)]
            m = (col >= lo) & (col < lo + R)
            safe = jnp.minimum(cur, CAP - 160)
            mi = jnp.where(m, jnp.int32(1), jnp.int32(0))
            pos = safe + plsc.cumsum(mi) - 1
            plsc.store_scatter(lcols, [pos], col - lo, mask=m)
            plsc.store_scatter(lrows, [pos], row, mask=m)
            cnt = jnp.sum(mi)
            return jnp.minimum(cur + cnt, CAP - 160)
        return lax.fori_loop(0, WE // LANE, group, cur)

    def pair(t, cur):
        wait(colw0, roww0, cs0, rs0)
        cur = process(colw0, roww0, cur)

        @pl.when(2 * t + 2 < NWIN)
        def _():
            issue(colw0, roww0, cs0, rs0, 2 * t + 2)

        wait(colw1, roww1, cs1, rs1)
        cur = process(colw1, roww1, cur)

        @pl.when(2 * t + 3 < NWIN)
        def _():
            issue(colw1, roww1, cs1, rs1, 2 * t + 3)
        return cur

    cur = lax.fori_loop(0, NWIN // 2, pair, jnp.int32(0))

    base = jnp.minimum(cur, CAP - 160)
    for g in range(K // LANE):
        lcols[pl.ds(base + g * LANE, LANE)] = jnp.full((LANE,), R, jnp.int32)
        lrows[pl.ds(base + g * LANE, LANE)] = jnp.zeros((LANE,), jnp.int32)
    cnt_final = ((base + K - 1) // K) * K

    pltpu.sync_copy(lrows, rows_o.at[wid])
    pltpu.sync_copy(lcols, cols_o.at[wid])
    cntv[pl.ds(0, LANE)] = jnp.full((LANE,), cnt_final, jnp.int32)
    pltpu.sync_copy(cntv, cnts_o.at[wid])

    iota = lax.broadcasted_iota(jnp.int32, (LANE,), 0)
    onehot0 = jnp.where(iota == 0, jnp.int32(1), jnp.int32(0))

    def zero_deg(i, _):
        degacc[pl.ds(i * LANE, LANE)] = jnp.zeros((LANE,), jnp.int32)
        return 0
    lax.fori_loop(0, R + 1, zero_deg, 0)

    def hist_group(g, _):
        cv = lcols[pl.ds(g * LANE, LANE)]
        for lane in range(LANE):
            c = jnp.sum(jnp.where(iota == lane, cv, jnp.int32(0)))
            plsc.addupdate(degacc.at[pl.ds(c * LANE, LANE)], onehot0)
        return 0
    lax.fori_loop(0, cnt_final // LANE, hist_group, 0)

    def compact(g, _):
        idx = (g * LANE + iota) * LANE
        degcomp[pl.ds(g * LANE, LANE)] = plsc.load_gather(degacc, [idx])
        return 0
    lax.fori_loop(0, R // LANE, compact, 0)
    pltpu.sync_copy(degcomp, deg_o.at[pl.ds(wid * R, R)])


def _conv_body(mp, rows, cols, cnts, out,
               acc, idx0, idx1, g0, g1, col0, col1, cntv,
               is0, is1, gs0, gs1, cs0, cs1):
    wid = _wid()
    iota = lax.broadcasted_iota(jnp.int32, (LANE,), 0)
    pltpu.sync_copy(cnts.at[wid], cntv)
    cv0 = cntv[pl.ds(0, LANE)]
    nw = jnp.sum(jnp.where(iota == 0, cv0, jnp.int32(0))) // K

    @plsc.parallel_loop(0, (R + 1) * HP // LANE, unroll=4)
    def zero(i):
        acc[pl.ds(i * LANE, LANE)] = jnp.zeros((LANE,), jnp.float32)

    def accum(g, colv):
        def body(gg, _):
            cv = colv[pl.ds(gg * LANE, LANE)]
            for lane in range(LANE):
                cs = jnp.take(cv, jnp.full((LANE,), lane, jnp.int32))
                base = cs * HP + iota
                j = gg * LANE + lane
                for k in range(NVEC):
                    plsc.addupdate_scatter(
                        acc, [base + k * LANE],
                        g[j, pl.ds(k * LANE, LANE)])
            return 0
        lax.fori_loop(0, K // LANE, body, 0)

    @pl.when(nw > 0)
    def _():
        pltpu.async_copy(rows.at[wid, pl.ds(0, K)], idx0, is0)
        pltpu.async_copy(cols.at[wid, pl.ds(0, K)], col0, cs0)

    @pl.when(nw > 1)
    def _():
        pltpu.async_copy(rows.at[wid, pl.ds(K, K)], idx1, is1)
        pltpu.async_copy(cols.at[wid, pl.ds(K, K)], col1, cs1)

    @pl.when(nw > 0)
    def _():
        pltpu.make_async_copy(rows.at[wid, pl.ds(0, K)], idx0, is0).wait()
        pltpu.async_copy(mp.at[idx0], g0, gs0)

    def pair(t, _):
        w0 = 2 * t
        w1 = 2 * t + 1
        pltpu.make_async_copy(mp.at[idx0], g0, gs0).wait()

        @pl.when(w0 + 2 < nw)
        def _():
            pltpu.async_copy(rows.at[wid, pl.ds((w0 + 2) * K, K)], idx0, is0)

        @pl.when(w1 < nw)
        def _():
            pltpu.make_async_copy(rows.at[wid, pl.ds(0, K)], idx1, is1).wait()
            pltpu.async_copy(mp.at[idx1], g1, gs1)

        pltpu.make_async_copy(cols.at[wid, pl.ds(0, K)], col0, cs0).wait()
        accum(g0, col0)

        @pl.when(w0 + 2 < nw)
        def _():
            pltpu.async_copy(cols.at[wid, pl.ds((w0 + 2) * K, K)], col0, cs0)

        @pl.when(w1 < nw)
        def _():
            pltpu.make_async_copy(mp.at[idx1], g1, gs1).wait()

            @pl.when(w1 + 2 < nw)
            def _():
                pltpu.async_copy(rows.at[wid, pl.ds((w1 + 2) * K, K)], idx1, is1)

            @pl.when(w0 + 2 < nw)
            def _():
                pltpu.make_async_copy(rows.at[wid, pl.ds(0, K)], idx0, is0).wait()
                pltpu.async_copy(mp.at[idx0], g0, gs0)

            pltpu.make_async_copy(cols.at[wid, pl.ds(0, K)], col1, cs1).wait()
            accum(g1, col1)

            @pl.when(w1 + 2 < nw)
            def _():
                pltpu.async_copy(cols.at[wid, pl.ds((w1 + 2) * K, K)], col1, cs1)
        return 0

    lax.fori_loop(0, (nw + 1) // 2, pair, 0)
    pltpu.sync_copy(acc.at[pl.ds(0, R * HP)], out.at[pl.ds(wid * R * HP, R * HP)])


def _embed_body(x_ref, emb_ref, w0_ref, deg_ref, h_ref, mp_ref, dinv_ref):
    xv = x_ref[...]
    ohi = lax.broadcasted_iota(jnp.int32, (NP, CP), 1)
    oh = (ohi == xv).astype(jnp.float32)
    h0 = jnp.dot(oh, emb_ref[...], preferred_element_type=jnp.float32)
    rmask = (lax.broadcasted_iota(jnp.int32, (NP, 1), 0) < NN).astype(jnp.float32)
    degf = deg_ref[...].astype(jnp.float32) + rmask
    dinv = rmask * lax.rsqrt(jnp.maximum(degf, 1.0))
    m0 = jnp.dot(h0, w0_ref[...], preferred_element_type=jnp.float32)
    h_ref[...] = h0
    mp_ref[...] = dinv * m0
    dinv_ref[...] = dinv


def _fuse_a_body(s_ref, mp_ref, dinv_ref, b_ref, hc_ref, st_ref):
    hc = dinv_ref[...] * (s_ref[...] + mp_ref[...]) + b_ref[...]
    s1 = jnp.sum(hc, axis=0, keepdims=True)
    mean = s1 * (1.0 / NN)
    rmask = (lax.broadcasted_iota(jnp.int32, (NP, 1), 0) < NN).astype(jnp.float32)
    dcen = (hc - mean) * rmask
    s2 = jnp.sum(dcen * dcen, axis=0, keepdims=True)
    var = s2 * (1.0 / NN)
    hc_ref[...] = hc
    st_ref[...] = jnp.concatenate([mean, lax.rsqrt(var + 1e-5)], axis=0)


def _make_fuse_b_body(last):
    def body(*refs):
        if last:
            (hc_ref, st_ref, h_ref, dinv_ref, g_ref, bt_ref, ho_ref) = refs
        else:
            (hc_ref, st_ref, h_ref, dinv_ref, g_ref, bt_ref, wn_ref,
             ho_ref, mpo_ref) = refs
        mean = st_ref[0:1, :]
        rstd = st_ref[1:2, :]
        hn = (hc_ref[...] - mean) * rstd * g_ref[...] + bt_ref[...]
        rmask = (lax.broadcasted_iota(jnp.int32, (NP, 1), 0) < NN).astype(jnp.float32)
        hr = jnp.maximum(hn, 0.0) * rmask + h_ref[...]
        ho_ref[...] = hr
        if not last:
            mpo_ref[...] = dinv_ref[...] * jnp.dot(
                hr, wn_ref[...], preferred_element_type=jnp.float32)
    return body


def _pool_body(h_ref, bc_ref, br_ref, w1_ref, b1_ref, w2_ref, b2_ref, o_ref):
    acc = jnp.zeros((GG, HP), jnp.float32)
    blkn = 1024
    for nb in range(NP // blkn):
        blk = h_ref[pl.ds(nb * blkn, blkn), :]
        bb = bc_ref[pl.ds(nb * blkn, blkn), :]
        bbr = br_ref[:, pl.ds(nb * blkn, blkn)]
        msk = (bb >= 0).astype(jnp.float32)
        lanei = lax.broadcasted_iota(jnp.int32, (blkn, HP), 1)
        blk2 = jnp.where(lanei == HP - 1, msk, blk)
        gi = lax.broadcasted_iota(jnp.int32, (GG, blkn), 0)
        p = (gi == bbr).astype(jnp.float32)
        acc = acc + jnp.dot(p, blk2, preferred_element_type=jnp.float32)
    counts = acc[:, HP - 1:HP]
    hg = acc / jnp.maximum(counts, 1.0)
    z = jnp.maximum(
        jnp.dot(hg, w1_ref[...], preferred_element_type=jnp.float32)
        + b1_ref[...], 0.0)
    o_ref[...] = jnp.dot(
        z, w2_ref[...], preferred_element_type=jnp.float32) + b2_ref[...]


def _tc(body, out_shape, *args):
    return pl.pallas_call(
        body, out_shape=out_shape,
        compiler_params=pltpu.CompilerParams(
            vmem_limit_bytes=120 * 1024 * 1024))(*args)


def kernel(x, edge_index, batch, emb, W, b, gamma, beta, W1, b1, W2, b2):
    f32 = jnp.float32
    i32 = jnp.int32
    x = x.astype(i32)
    edge_index = edge_index.astype(i32)
    batch = batch.astype(i32)

    embp = jnp.zeros((CP, HP), f32).at[:CC, :HH].set(emb)
    Wp = jnp.zeros((LL, HP, HP), f32).at[:, :HH, :HH].set(W)
    bp = jnp.zeros((LL, 1, HP), f32).at[:, 0, :HH].set(b)
    gp = jnp.zeros((LL, 1, HP), f32).at[:, 0, :HH].set(gamma)
    btp = jnp.zeros((LL, 1, HP), f32).at[:, 0, :HH].set(beta)
    W1p = jnp.zeros((HP, HHP), f32).at[:HH, :H2].set(W1)
    b1p = jnp.zeros((1, HHP), f32).at[0, :H2].set(b1)
    W2p = jnp.zeros((HHP, 128), f32).at[:H2, 0].set(W2[:, 0])
    b2p = jnp.zeros((1, 128), f32).at[0, 0].set(b2[0])
    xp = jnp.full((NP, 1), -1, i32).at[:NN].set(x)
    batc = jnp.full((NP, 1), -1, i32).at[:NN, 0].set(batch)
    batr = batc.reshape(1, NP)

    prep_k, conv_k = _sc_kernels()
    rows, colsl, cnts, deg = prep_k(edge_index[0], edge_index[1])

    h, mp, dinv = _tc(
        _embed_body,
        (jax.ShapeDtypeStruct((NP, HP), f32),
         jax.ShapeDtypeStruct((NP, HP), f32),
         jax.ShapeDtypeStruct((NP, 1), f32)),
        xp, embp, Wp[0], deg.reshape(NP, 1))

    for i in range(LL):
        s = conv_k(mp, rows, colsl, cnts).reshape(NP, HP)
        hc, st = _tc(
            _fuse_a_body,
            (jax.ShapeDtypeStruct((NP, HP), f32),
             jax.ShapeDtypeStruct((2, HP), f32)),
            s, mp, dinv, bp[i])
        if i < LL - 1:
            h, mp = _tc(
                _make_fuse_b_body(False),
                (jax.ShapeDtypeStruct((NP, HP), f32),
                 jax.ShapeDtypeStruct((NP, HP), f32)),
                hc, st, h, dinv, gp[i], btp[i], Wp[i + 1])
        else:
            h = _tc(
                _make_fuse_b_body(True),
                jax.ShapeDtypeStruct((NP, HP), f32),
                hc, st, h, dinv, gp[i], btp[i])

    o = _tc(
        _pool_body,
        jax.ShapeDtypeStruct((GG, 128), f32),
        h, batc, batr, W1p, b1p, W2p, b2p)
    return o[:, :1]


# speedup vs baseline: 1.5302x; 1.5302x over previous
import functools

import jax
import jax.numpy as jnp
from jax import lax
from jax.experimental import pallas as pl
from jax.experimental.pallas import tpu as pltpu
from jax.experimental.pallas import tpu_sc as plsc

NN = 10000
EE = 320000
HH = 145
LL = 4
CC = 21
GG = 512
H2 = 72

NC, NS, LANE = 2, 16, 16
NWK = NC * NS
R = 320
NP = NWK * R
HP = 160
CP = 32
HHP = 128
CAP = 16384
WE = 2000
NWIN = EE // WE
K = 128
RP = 336
NVEC = HP // LANE

def _wid():
    return lax.axis_index("s") * NC + lax.axis_index("c")


@functools.cache
def _sc_kernels():
    mesh = plsc.VectorSubcoreMesh(
        core_axis_name="c", subcore_axis_name="s",
        num_cores=NC, num_subcores=NS)
    cparams = pltpu.CompilerParams(needs_layout_passes=False,
                                   use_tc_tiling_on_sc=False)
    prep = pl.kernel(
        _prep_body,
        compiler_params=cparams,
        out_type=[
            jax.ShapeDtypeStruct((NWK, CAP), jnp.int32),
            jax.ShapeDtypeStruct((NWK, RP), jnp.int32),
            jax.ShapeDtypeStruct((NWK, 16), jnp.int32),
            jax.ShapeDtypeStruct((NP,), jnp.int32),
        ],
        mesh=mesh,
        scratch_types=[
            pltpu.VMEM((WE,), jnp.int32),
            pltpu.VMEM((WE,), jnp.int32),
            pltpu.VMEM((WE,), jnp.int32),
            pltpu.VMEM((WE,), jnp.int32),
            pltpu.VMEM((CAP,), jnp.int32),
            pltpu.VMEM((CAP,), jnp.int32),
            pltpu.VMEM((CAP,), jnp.int32),
            pltpu.VMEM((RP * 16,), jnp.int32),
            pltpu.VMEM((RP,), jnp.int32),
            pltpu.VMEM((RP,), jnp.int32),
            pltpu.VMEM((R,), jnp.int32),
            pltpu.VMEM((16,), jnp.int32),
            pltpu.SemaphoreType.DMA,
            pltpu.SemaphoreType.DMA,
            pltpu.SemaphoreType.DMA,
            pltpu.SemaphoreType.DMA,
        ],
    )
    conv = pl.kernel(
        _conv_body,
        compiler_params=cparams,
        out_type=jax.ShapeDtypeStruct((NP * HP,), jnp.float32),
        mesh=mesh,
        scratch_types=[
            pltpu.VMEM(((R + 1) * HP,), jnp.float32),
            pltpu.VMEM((K,), jnp.int32),
            pltpu.VMEM((K,), jnp.int32),
            pltpu.VMEM((K, HP), jnp.float32),
            pltpu.VMEM((K, HP), jnp.float32),
            pltpu.VMEM((RP,), jnp.int32),
            pltpu.VMEM((16,), jnp.int32),
            pltpu.SemaphoreType.DMA,
            pltpu.SemaphoreType.DMA,
            pltpu.SemaphoreType.DMA,
            pltpu.SemaphoreType.DMA,
        ],
    )
    return prep, conv


def _prep_body(erow, ecol, rows_o, rp_o, cnts_o, deg_o,
               colw0, roww0, colw1, roww1, lrows, lcols, srows,
               degacc, rp, cursor, degcomp, cntv, cs0, rs0, cs1, rs1):
    wid = _wid()
    lo = wid * R

    def issue(slot_col, slot_row, csem, rsem, w):
        pltpu.async_copy(ecol.at[pl.ds(w * WE, WE)], slot_col, csem)
        pltpu.async_copy(erow.at[pl.ds(w * WE, WE)], slot_row, rsem)

    def wait(slot_col, slot_row, csem, rsem):
        pltpu.make_async_copy(ecol.at[pl.ds(0, WE)], slot_col, csem).wait()
        pltpu.make_async_copy(erow.at[pl.ds(0, WE)], slot_row, rsem).wait()

    issue(colw0, roww0, cs0, rs0, 0)
    issue(colw1, roww1, cs1, rs1, 1)

    def process(colw, roww, cur):
        def group(g, cur):
            col = colw[pl.ds(g * LANE, LANE)]
            row = roww[pl.ds(g * LANE, LANE)]
            m = (col >= lo) & (col < lo + R)
            safe = jnp.minimum(cur, CAP - 160)
            mi = jnp.where(m, jnp.int32(1), jnp.int32(0))
            pos = safe + plsc.cumsum(mi) - 1
            plsc.store_scatter(lcols, [pos], col - lo, mask=m)
            plsc.store_scatter(lrows, [pos], row, mask=m)
            cnt = jnp.sum(mi)
            return jnp.minimum(cur + cnt, CAP - 160)
        return lax.fori_loop(0, WE // LANE, group, cur)

    def pair(t, cur):
        wait(colw0, roww0, cs0, rs0)
        cur = process(colw0, roww0, cur)

        @pl.when(2 * t + 2 < NWIN)
        def _():
            issue(colw0, roww0, cs0, rs0, 2 * t + 2)

        wait(colw1, roww1, cs1, rs1)
        cur = process(colw1, roww1, cur)

        @pl.when(2 * t + 3 < NWIN)
        def _():
            issue(colw1, roww1, cs1, rs1, 2 * t + 3)
        return cur

    cur = lax.fori_loop(0, NWIN // 2, pair, jnp.int32(0))

    base = jnp.minimum(cur, CAP - 160)
    for g in range(K // LANE):
        lcols[pl.ds(base + g * LANE, LANE)] = jnp.full((LANE,), R, jnp.int32)
        lrows[pl.ds(base + g * LANE, LANE)] = jnp.zeros((LANE,), jnp.int32)
    cnt_final = ((base + K - 1) // K) * K

    cntv[pl.ds(0, LANE)] = jnp.full((LANE,), cnt_final, jnp.int32)
    pltpu.sync_copy(cntv, cnts_o.at[wid])

    iota = lax.broadcasted_iota(jnp.int32, (LANE,), 0)
    onehot0 = jnp.where(iota == 0, jnp.int32(1), jnp.int32(0))
    lane0 = iota == 0

    def zero_deg(i, _):
        degacc[pl.ds(i * LANE, LANE)] = jnp.zeros((LANE,), jnp.int32)
        return 0
    lax.fori_loop(0, RP, zero_deg, 0)

    def hist_group(g, _):
        cv = lcols[pl.ds(g * LANE, LANE)]
        for lane in range(LANE):
            c = jnp.sum(jnp.where(iota == lane, cv, jnp.int32(0)))
            plsc.addupdate(degacc.at[pl.ds(c * LANE, LANE)], onehot0)
        return 0
    lax.fori_loop(0, cnt_final // LANE, hist_group, 0)

    def compact(g, _):
        idx = (g * LANE + iota) * LANE
        degcomp[pl.ds(g * LANE, LANE)] = plsc.load_gather(degacc, [idx])
        return 0
    lax.fori_loop(0, R // LANE, compact, 0)
    pltpu.sync_copy(degcomp, deg_o.at[pl.ds(wid * R, R)])

    def prefix(g, tot):
        v = plsc.load_gather(degacc, [(g * LANE + iota) * LANE])
        ex = plsc.cumsum(v) - v
        rp[pl.ds(g * LANE, LANE)] = ex + tot
        cursor[pl.ds(g * LANE, LANE)] = ex + tot
        return tot + jnp.sum(v)
    lax.fori_loop(0, RP // LANE, prefix, jnp.int32(0))
    pltpu.sync_copy(rp, rp_o.at[wid])

    def permute(e, _):
        eb = (e // LANE) * LANE
        lane = e - eb
        lanev = iota * 0 + lane
        cv = lcols[pl.ds(eb, LANE)]
        rv = lrows[pl.ds(eb, LANE)]
        cs = jnp.take(cv, lanev)
        rs = jnp.take(rv, lanev)
        pos = plsc.load_gather(cursor, [cs])
        plsc.store_scatter(cursor, [cs], pos + 1, mask=lane0)
        plsc.store_scatter(srows, [pos], rs, mask=lane0)
        return 0
    lax.fori_loop(0, cnt_final, permute, 0)
    pltpu.sync_copy(srows, rows_o.at[wid])


def _conv_body(mp, rows, rp_h, cnts, out,
               acc, idx0, idx1, g0, g1, rpv, cntv,
               is0, is1, gs0, gs1):
    wid = _wid()
    iota = lax.broadcasted_iota(jnp.int32, (LANE,), 0)
    pltpu.sync_copy(cnts.at[wid], cntv)
    cv0 = cntv[pl.ds(0, LANE)]
    nw = jnp.sum(jnp.where(iota == 0, cv0, jnp.int32(0))) // K
    pltpu.sync_copy(rp_h.at[wid], rpv)

    @plsc.parallel_loop(0, (R + 1) * HP // LANE, unroll=4)
    def zero(i):
        acc[pl.ds(i * LANE, LANE)] = jnp.zeros((LANE,), jnp.float32)

    def rp_at(c):
        v = rpv[pl.ds((c // LANE) * LANE, LANE)]
        return jnp.sum(jnp.where(iota == (c % LANE), v, jnp.int32(0)))

    def accum(g, w, c0):
        e0 = w * K
        e1 = e0 + K

        def row_cond(st):
            c, rpc = st
            return (c < R) & (rpc < e1)

        def row_body(st):
            c, rpc = st
            rpn = rp_at(c + 1)
            s = jnp.maximum(rpc, e0)
            t = jnp.minimum(rpn, e1)

            def edge(e, regs):
                je = e - e0
                return tuple(
                    regs[k] + g[je, pl.ds(k * LANE, LANE)]
                    for k in range(NVEC))
            regs = lax.fori_loop(
                s, t, edge,
                tuple(jnp.zeros((LANE,), jnp.float32) for _ in range(NVEC)))
            base = c * HP
            for k in range(NVEC):
                plsc.addupdate(acc.at[pl.ds(base + k * LANE, LANE)], regs[k])
            return (c + 1, rpn)

        c_ex, rp_ex = lax.while_loop(row_cond, row_body, (c0, rp_at(c0)))
        return jnp.where(rp_ex > e1, c_ex - 1, c_ex)

    @pl.when(nw > 0)
    def _():
        pltpu.async_copy(rows.at[wid, pl.ds(0, K)], idx0, is0)

    @pl.when(nw > 1)
    def _():
        pltpu.async_copy(rows.at[wid, pl.ds(K, K)], idx1, is1)

    @pl.when(nw > 0)
    def _():
        pltpu.make_async_copy(rows.at[wid, pl.ds(0, K)], idx0, is0).wait()
        pltpu.async_copy(mp.at[idx0], g0, gs0)

    def pair(t, c0):
        w0 = 2 * t
        w1 = 2 * t + 1
        pltpu.make_async_copy(mp.at[idx0], g0, gs0).wait()

        @pl.when(w0 + 2 < nw)
        def _():
            pltpu.async_copy(rows.at[wid, pl.ds((w0 + 2) * K, K)], idx0, is0)

        @pl.when(w1 < nw)
        def _():
            pltpu.make_async_copy(rows.at[wid, pl.ds(0, K)], idx1, is1).wait()
            pltpu.async_copy(mp.at[idx1], g1, gs1)

        c0 = accum(g0, w0, c0)

        def snd():
            pltpu.make_async_copy(mp.at[idx1], g1, gs1).wait()

            @pl.when(w1 + 2 < nw)
            def _():
                pltpu.async_copy(rows.at[wid, pl.ds((w1 + 2) * K, K)], idx1, is1)

            @pl.when(w0 + 2 < nw)
            def _():
                pltpu.make_async_copy(rows.at[wid, pl.ds(0, K)], idx0, is0).wait()
                pltpu.async_copy(mp.at[idx0], g0, gs0)

            return accum(g1, w1, c0)

        return lax.cond(w1 < nw, snd, lambda: c0)

    lax.fori_loop(0, (nw + 1) // 2, pair, jnp.int32(0))
    pltpu.sync_copy(acc.at[pl.ds(0, R * HP)], out.at[pl.ds(wid * R * HP, R * HP)])


def _embed_body(x_ref, emb_ref, w0_ref, deg_ref, h_ref, mp_ref, dinv_ref):
    xv = x_ref[...]
    ohi = lax.broadcasted_iota(jnp.int32, (NP, CP), 1)
    oh = (ohi == xv).astype(jnp.float32)
    h0 = jnp.dot(oh, emb_ref[...], preferred_element_type=jnp.float32)
    rmask = (lax.broadcasted_iota(jnp.int32, (NP, 1), 0) < NN).astype(jnp.float32)
    degf = deg_ref[...].astype(jnp.float32) + rmask
    dinv = rmask * lax.rsqrt(jnp.maximum(degf, 1.0))
    m0 = jnp.dot(h0, w0_ref[...], preferred_element_type=jnp.float32)
    h_ref[...] = h0
    mp_ref[...] = dinv * m0
    dinv_ref[...] = dinv


def _fuse_a_body(s_ref, mp_ref, dinv_ref, b_ref, hc_ref, st_ref):
    hc = dinv_ref[...] * (s_ref[...] + mp_ref[...]) + b_ref[...]
    s1 = jnp.sum(hc, axis=0, keepdims=True)
    mean = s1 * (1.0 / NN)
    rmask = (lax.broadcasted_iota(jnp.int32, (NP, 1), 0) < NN).astype(jnp.float32)
    dcen = (hc - mean) * rmask
    s2 = jnp.sum(dcen * dcen, axis=0, keepdims=True)
    var = s2 * (1.0 / NN)
    hc_ref[...] = hc
    st_ref[...] = jnp.concatenate([mean, lax.rsqrt(var + 1e-5)], axis=0)


def _make_fuse_b_body(last):
    def body(*refs):
        if last:
            (hc_ref, st_ref, h_ref, dinv_ref, g_ref, bt_ref, ho_ref) = refs
        else:
            (hc_ref, st_ref, h_ref, dinv_ref, g_ref, bt_ref, wn_ref,
             ho_ref, mpo_ref) = refs
        mean = st_ref[0:1, :]
        rstd = st_ref[1:2, :]
        hn = (hc_ref[...] - mean) * rstd * g_ref[...] + bt_ref[...]
        rmask = (lax.broadcasted_iota(jnp.int32, (NP, 1), 0) < NN).astype(jnp.float32)
        hr = jnp.maximum(hn, 0.0) * rmask + h_ref[...]
        ho_ref[...] = hr
        if not last:
            mpo_ref[...] = dinv_ref[...] * jnp.dot(
                hr, wn_ref[...], preferred_element_type=jnp.float32)
    return body


def _pool_body(h_ref, bc_ref, br_ref, w1_ref, b1_ref, w2_ref, b2_ref, o_ref):
    acc = jnp.zeros((GG, HP), jnp.float32)
    blkn = 1024
    for nb in range(NP // blkn):
        blk = h_ref[pl.ds(nb * blkn, blkn), :]
        bb = bc_ref[pl.ds(nb * blkn, blkn), :]
        bbr = br_ref[:, pl.ds(nb * blkn, blkn)]
        msk = (bb >= 0).astype(jnp.float32)
        lanei = lax.broadcasted_iota(jnp.int32, (blkn, HP), 1)
        blk2 = jnp.where(lanei == HP - 1, msk, blk)
        gi = lax.broadcasted_iota(jnp.int32, (GG, blkn), 0)
        p = (gi == bbr).astype(jnp.float32)
        acc = acc + jnp.dot(p, blk2, preferred_element_type=jnp.float32)
    counts = acc[:, HP - 1:HP]
    hg = acc / jnp.maximum(counts, 1.0)
    z = jnp.maximum(
        jnp.dot(hg, w1_ref[...], preferred_element_type=jnp.float32)
        + b1_ref[...], 0.0)
    o_ref[...] = jnp.dot(
        z, w2_ref[...], preferred_element_type=jnp.float32) + b2_ref[...]


def _tc(body, out_shape, *args):
    return pl.pallas_call(
        body, out_shape=out_shape,
        compiler_params=pltpu.CompilerParams(
            vmem_limit_bytes=120 * 1024 * 1024))(*args)


def kernel(x, edge_index, batch, emb, W, b, gamma, beta, W1, b1, W2, b2):
    f32 = jnp.float32
    i32 = jnp.int32
    x = x.astype(i32)
    edge_index = edge_index.astype(i32)
    batch = batch.astype(i32)

    embp = jnp.zeros((CP, HP), f32).at[:CC, :HH].set(emb)
    Wp = jnp.zeros((LL, HP, HP), f32).at[:, :HH, :HH].set(W)
    bp = jnp.zeros((LL, 1, HP), f32).at[:, 0, :HH].set(b)
    gp = jnp.zeros((LL, 1, HP), f32).at[:, 0, :HH].set(gamma)
    btp = jnp.zeros((LL, 1, HP), f32).at[:, 0, :HH].set(beta)
    W1p = jnp.zeros((HP, HHP), f32).at[:HH, :H2].set(W1)
    b1p = jnp.zeros((1, HHP), f32).at[0, :H2].set(b1)
    W2p = jnp.zeros((HHP, 128), f32).at[:H2, 0].set(W2[:, 0])
    b2p = jnp.zeros((1, 128), f32).at[0, 0].set(b2[0])
    xp = jnp.full((NP, 1), -1, i32).at[:NN].set(x)
    batc = jnp.full((NP, 1), -1, i32).at[:NN, 0].set(batch)
    batr = batc.reshape(1, NP)

    prep_k, conv_k = _sc_kernels()
    rows, rptr, cnts, deg = prep_k(edge_index[0], edge_index[1])

    h, mp, dinv = _tc(
        _embed_body,
        (jax.ShapeDtypeStruct((NP, HP), f32),
         jax.ShapeDtypeStruct((NP, HP), f32),
         jax.ShapeDtypeStruct((NP, 1), f32)),
        xp, embp, Wp[0], deg.reshape(NP, 1))

    for i in range(LL):
        s = conv_k(mp, rows, rptr, cnts).reshape(NP, HP)
        hc, st = _tc(
            _fuse_a_body,
            (jax.ShapeDtypeStruct((NP, HP), f32),
             jax.ShapeDtypeStruct((2, HP), f32)),
            s, mp, dinv, bp[i])
        if i < LL - 1:
            h, mp = _tc(
                _make_fuse_b_body(False),
                (jax.ShapeDtypeStruct((NP, HP), f32),
                 jax.ShapeDtypeStruct((NP, HP), f32)),
                hc, st, h, dinv, gp[i], btp[i], Wp[i + 1])
        else:
            h = _tc(
                _make_fuse_b_body(True),
                jax.ShapeDtypeStruct((NP, HP), f32),
                hc, st, h, dinv, gp[i], btp[i])

    o = _tc(
        _pool_body,
        jax.ShapeDtypeStruct((GG, 128), f32),
        h, batc, batr, W1p, b1p, W2p, b2p)
    return o[:, :1]


# speedup vs baseline: 1.6480x; 1.0770x over previous
import functools

import jax
import jax.numpy as jnp
from jax import lax
from jax.experimental import pallas as pl
from jax.experimental.pallas import tpu as pltpu
from jax.experimental.pallas import tpu_sc as plsc

NN = 10000
EE = 320000
HH = 145
LL = 4
CC = 21
GG = 512
H2 = 72

NC, NS, LANE = 2, 16, 16
NWK = NC * NS
R = 320
NP = NWK * R
HP = 160
CP = 32
HHP = 128
CAP = 16384
WE = 2000
NWIN = EE // WE
K = 128
RP = 336
NVEC = HP // LANE

def _wid():
    return lax.axis_index("s") * NC + lax.axis_index("c")


@functools.cache
def _sc_kernels():
    mesh = plsc.VectorSubcoreMesh(
        core_axis_name="c", subcore_axis_name="s",
        num_cores=NC, num_subcores=NS)
    cparams = pltpu.CompilerParams(needs_layout_passes=False,
                                   use_tc_tiling_on_sc=False)
    prep = pl.kernel(
        _prep_body,
        compiler_params=cparams,
        out_type=[
            jax.ShapeDtypeStruct((NWK, CAP), jnp.int32),
            jax.ShapeDtypeStruct((NWK, RP), jnp.int32),
            jax.ShapeDtypeStruct((NWK, 16), jnp.int32),
            jax.ShapeDtypeStruct((NP,), jnp.int32),
        ],
        mesh=mesh,
        scratch_types=[
            pltpu.VMEM((WE,), jnp.int32),
            pltpu.VMEM((WE,), jnp.int32),
            pltpu.VMEM((WE,), jnp.int32),
            pltpu.VMEM((WE,), jnp.int32),
            pltpu.VMEM((CAP,), jnp.int32),
            pltpu.VMEM((CAP,), jnp.int32),
            pltpu.VMEM((CAP,), jnp.int32),
            pltpu.VMEM((RP * 16,), jnp.int32),
            pltpu.VMEM((RP,), jnp.int32),
            pltpu.VMEM((RP,), jnp.int32),
            pltpu.VMEM((R,), jnp.int32),
            pltpu.VMEM((16,), jnp.int32),
            pltpu.SemaphoreType.DMA,
            pltpu.SemaphoreType.DMA,
            pltpu.SemaphoreType.DMA,
            pltpu.SemaphoreType.DMA,
        ],
    )
    conv = pl.kernel(
        _conv_body,
        compiler_params=cparams,
        out_type=jax.ShapeDtypeStruct((NP * HP,), jnp.float32),
        mesh=mesh,
        scratch_types=[
            pltpu.VMEM(((R + 1) * HP,), jnp.float32),
            pltpu.VMEM((K,), jnp.int32),
            pltpu.VMEM((K,), jnp.int32),
            pltpu.VMEM((K, HP), jnp.float32),
            pltpu.VMEM((K, HP), jnp.float32),
            pltpu.VMEM((RP,), jnp.int32),
            pltpu.VMEM((16,), jnp.int32),
            pltpu.SemaphoreType.DMA,
            pltpu.SemaphoreType.DMA,
            pltpu.SemaphoreType.DMA,
            pltpu.SemaphoreType.DMA,
        ],
    )
    return prep, conv


def _prep_body(erow, ecol, rows_o, rp_o, cnts_o, deg_o,
               colw0, roww0, colw1, roww1, lrows, lcols, srows,
               degacc, rp, cursor, degcomp, cntv, cs0, rs0, cs1, rs1):
    wid = _wid()
    lo = wid * R

    def issue(slot_col, slot_row, csem, rsem, w):
        pltpu.async_copy(ecol.at[pl.ds(w * WE, WE)], slot_col, csem)
        pltpu.async_copy(erow.at[pl.ds(w * WE, WE)], slot_row, rsem)

    def wait(slot_col, slot_row, csem, rsem):
        pltpu.make_async_copy(ecol.at[pl.ds(0, WE)], slot_col, csem).wait()
        pltpu.make_async_copy(erow.at[pl.ds(0, WE)], slot_row, rsem).wait()

    issue(colw0, roww0, cs0, rs0, 0)
    issue(colw1, roww1, cs1, rs1, 1)

    def process(colw, roww, cur):
        def group(g, cur):
            col = colw[pl.ds(g * LANE, LANE)]
            row = roww[pl.ds(g * LANE, LANE)]
            m = (col >= lo) & (col < lo + R)
            safe = jnp.minimum(cur, CAP - 160)
            mi = jnp.where(m, jnp.int32(1), jnp.int32(0))
            pos = safe + plsc.cumsum(mi) - 1
            plsc.store_scatter(lcols, [pos], col - lo, mask=m)
            plsc.store_scatter(lrows, [pos], row, mask=m)
            cnt = jnp.sum(mi)
            return jnp.minimum(cur + cnt, CAP - 160)
        return lax.fori_loop(0, WE // LANE, group, cur)

    def pair(t, cur):
        wait(colw0, roww0, cs0, rs0)
        cur = process(colw0, roww0, cur)

        @pl.when(2 * t + 2 < NWIN)
        def _():
            issue(colw0, roww0, cs0, rs0, 2 * t + 2)

        wait(colw1, roww1, cs1, rs1)
        cur = process(colw1, roww1, cur)

        @pl.when(2 * t + 3 < NWIN)
        def _():
            issue(colw1, roww1, cs1, rs1, 2 * t + 3)
        return cur

    cur = lax.fori_loop(0, NWIN // 2, pair, jnp.int32(0))

    base = jnp.minimum(cur, CAP - 160)
    for g in range(K // LANE):
        lcols[pl.ds(base + g * LANE, LANE)] = jnp.full((LANE,), R, jnp.int32)
        lrows[pl.ds(base + g * LANE, LANE)] = jnp.zeros((LANE,), jnp.int32)
    cnt_final = ((base + K - 1) // K) * K

    cntv[pl.ds(0, LANE)] = jnp.full((LANE,), cnt_final, jnp.int32)
    pltpu.sync_copy(cntv, cnts_o.at[wid])

    iota = lax.broadcasted_iota(jnp.int32, (LANE,), 0)
    onehot0 = jnp.where(iota == 0, jnp.int32(1), jnp.int32(0))
    lane0 = iota == 0

    def zero_deg(i, _):
        degacc[pl.ds(i * LANE, LANE)] = jnp.zeros((LANE,), jnp.int32)
        return 0
    lax.fori_loop(0, RP, zero_deg, 0)

    def hist_group(g, _):
        cv = lcols[pl.ds(g * LANE, LANE)]
        for lane in range(LANE):
            c = jnp.sum(jnp.where(iota == lane, cv, jnp.int32(0)))
            plsc.addupdate(degacc.at[pl.ds(c * LANE, LANE)], onehot0)
        return 0
    lax.fori_loop(0, cnt_final // LANE, hist_group, 0)

    def compact(g, _):
        idx = (g * LANE + iota) * LANE
        degcomp[pl.ds(g * LANE, LANE)] = plsc.load_gather(degacc, [idx])
        return 0
    lax.fori_loop(0, R // LANE, compact, 0)
    pltpu.sync_copy(degcomp, deg_o.at[pl.ds(wid * R, R)])

    def prefix(g, tot):
        v = plsc.load_gather(degacc, [(g * LANE + iota) * LANE])
        ex = plsc.cumsum(v) - v
        rp[pl.ds(g * LANE, LANE)] = ex + tot
        cursor[pl.ds(g * LANE, LANE)] = ex + tot
        return tot + jnp.sum(v)
    lax.fori_loop(0, RP // LANE, prefix, jnp.int32(0))
    pltpu.sync_copy(rp, rp_o.at[wid])

    def permute(g, _):
        cv = lcols[pl.ds(g * LANE, LANE)]
        rv = lrows[pl.ds(g * LANE, LANE)]
        cs, rs = plsc.sort_key_val(cv, rv)
        cprev = jnp.take(cs, jnp.maximum(iota - 1, 0))
        same = (iota > 0) & (cs == cprev)
        brk = jnp.where(same, jnp.int32(0), iota)
        rank = iota - plsc.cummax(brk)
        cnext = jnp.take(cs, jnp.minimum(iota + 1, LANE - 1))
        last = (iota == LANE - 1) | (cs != cnext)
        pos = plsc.load_gather(cursor, [cs]) + rank
        plsc.store_scatter(cursor, [cs], pos + 1, mask=last)
        plsc.store_scatter(srows, [pos], rs)
        return 0
    lax.fori_loop(0, cnt_final // LANE, permute, 0)
    pltpu.sync_copy(srows, rows_o.at[wid])


def _conv_body(mp, rows, rp_h, cnts, out,
               acc, idx0, idx1, g0, g1, rpv, cntv,
               is0, is1, gs0, gs1):
    wid = _wid()
    iota = lax.broadcasted_iota(jnp.int32, (LANE,), 0)
    pltpu.sync_copy(cnts.at[wid], cntv)
    cv0 = cntv[pl.ds(0, LANE)]
    nw = jnp.sum(jnp.where(iota == 0, cv0, jnp.int32(0))) // K
    pltpu.sync_copy(rp_h.at[wid], rpv)

    @plsc.parallel_loop(0, (R + 1) * HP // LANE, unroll=4)
    def zero(i):
        acc[pl.ds(i * LANE, LANE)] = jnp.zeros((LANE,), jnp.float32)

    def rp_at(c):
        v = rpv[pl.ds((c // LANE) * LANE, LANE)]
        return jnp.sum(jnp.where(iota == (c % LANE), v, jnp.int32(0)))

    def accum(g, w, c0):
        e0 = w * K
        e1 = e0 + K

        def row_cond(st):
            c, rpc = st
            return (c < R) & (rpc < e1)

        def row_body(st):
            c, rpc = st
            rpn = rp_at(c + 1)
            s = jnp.maximum(rpc, e0)
            t = jnp.minimum(rpn, e1)

            def edge2(i, regs):
                je = s - e0 + 2 * i
                return tuple(
                    regs[k] + g[je, pl.ds(k * LANE, LANE)]
                    + g[je + 1, pl.ds(k * LANE, LANE)]
                    for k in range(NVEC))
            n = t - s
            regs = lax.fori_loop(
                0, n // 2, edge2,
                tuple(jnp.zeros((LANE,), jnp.float32) for _ in range(NVEC)))

            def tail(regs):
                je = t - 1 - e0
                return tuple(
                    regs[k] + g[je, pl.ds(k * LANE, LANE)]
                    for k in range(NVEC))
            regs = lax.cond(n % 2 == 1, tail, lambda r: r, regs)
            base = c * HP
            for k in range(NVEC):
                plsc.addupdate(acc.at[pl.ds(base + k * LANE, LANE)], regs[k])
            return (c + 1, rpn)

        c_ex, rp_ex = lax.while_loop(row_cond, row_body, (c0, rp_at(c0)))
        return jnp.where(rp_ex > e1, c_ex - 1, c_ex)

    @pl.when(nw > 0)
    def _():
        pltpu.async_copy(rows.at[wid, pl.ds(0, K)], idx0, is0)

    @pl.when(nw > 1)
    def _():
        pltpu.async_copy(rows.at[wid, pl.ds(K, K)], idx1, is1)

    @pl.when(nw > 0)
    def _():
        pltpu.make_async_copy(rows.at[wid, pl.ds(0, K)], idx0, is0).wait()
        pltpu.async_copy(mp.at[idx0], g0, gs0)

    def pair(t, c0):
        w0 = 2 * t
        w1 = 2 * t + 1
        pltpu.make_async_copy(mp.at[idx0], g0, gs0).wait()

        @pl.when(w0 + 2 < nw)
        def _():
            pltpu.async_copy(rows.at[wid, pl.ds((w0 + 2) * K, K)], idx0, is0)

        @pl.when(w1 < nw)
        def _():
            pltpu.make_async_copy(rows.at[wid, pl.ds(0, K)], idx1, is1).wait()
            pltpu.async_copy(mp.at[idx1], g1, gs1)

        c0 = accum(g0, w0, c0)

        def snd():
            pltpu.make_async_copy(mp.at[idx1], g1, gs1).wait()

            @pl.when(w1 + 2 < nw)
            def _():
                pltpu.async_copy(rows.at[wid, pl.ds((w1 + 2) * K, K)], idx1, is1)

            @pl.when(w0 + 2 < nw)
            def _():
                pltpu.make_async_copy(rows.at[wid, pl.ds(0, K)], idx0, is0).wait()
                pltpu.async_copy(mp.at[idx0], g0, gs0)

            return accum(g1, w1, c0)

        return lax.cond(w1 < nw, snd, lambda: c0)

    lax.fori_loop(0, (nw + 1) // 2, pair, jnp.int32(0))
    pltpu.sync_copy(acc.at[pl.ds(0, R * HP)], out.at[pl.ds(wid * R * HP, R * HP)])


def _embed_body(x_ref, emb_ref, w0_ref, deg_ref, h_ref, mp_ref, dinv_ref):
    xv = x_ref[...]
    ohi = lax.broadcasted_iota(jnp.int32, (NP, CP), 1)
    oh = (ohi == xv).astype(jnp.float32)
    h0 = jnp.dot(oh, emb_ref[...], preferred_element_type=jnp.float32)
    rmask = (lax.broadcasted_iota(jnp.int32, (NP, 1), 0) < NN).astype(jnp.float32)
    degf = deg_ref[...].astype(jnp.float32) + rmask
    dinv = rmask * lax.rsqrt(jnp.maximum(degf, 1.0))
    m0 = jnp.dot(h0, w0_ref[...], preferred_element_type=jnp.float32)
    h_ref[...] = h0
    mp_ref[...] = dinv * m0
    dinv_ref[...] = dinv


def _fuse_a_body(s_ref, mp_ref, dinv_ref, b_ref, hc_ref, st_ref):
    hc = dinv_ref[...] * (s_ref[...] + mp_ref[...]) + b_ref[...]
    s1 = jnp.sum(hc, axis=0, keepdims=True)
    mean = s1 * (1.0 / NN)
    rmask = (lax.broadcasted_iota(jnp.int32, (NP, 1), 0) < NN).astype(jnp.float32)
    dcen = (hc - mean) * rmask
    s2 = jnp.sum(dcen * dcen, axis=0, keepdims=True)
    var = s2 * (1.0 / NN)
    hc_ref[...] = hc
    st_ref[...] = jnp.concatenate([mean, lax.rsqrt(var + 1e-5)], axis=0)


def _make_fuse_b_body(last):
    def body(*refs):
        if last:
            (hc_ref, st_ref, h_ref, dinv_ref, g_ref, bt_ref, ho_ref) = refs
        else:
            (hc_ref, st_ref, h_ref, dinv_ref, g_ref, bt_ref, wn_ref,
             ho_ref, mpo_ref) = refs
        mean = st_ref[0:1, :]
        rstd = st_ref[1:2, :]
        hn = (hc_ref[...] - mean) * rstd * g_ref[...] + bt_ref[...]
        rmask = (lax.broadcasted_iota(jnp.int32, (NP, 1), 0) < NN).astype(jnp.float32)
        hr = jnp.maximum(hn, 0.0) * rmask + h_ref[...]
        ho_ref[...] = hr
        if not last:
            mpo_ref[...] = dinv_ref[...] * jnp.dot(
                hr, wn_ref[...], preferred_element_type=jnp.float32)
    return body


def _pool_body(h_ref, bc_ref, br_ref, w1_ref, b1_ref, w2_ref, b2_ref, o_ref):
    acc = jnp.zeros((GG, HP), jnp.float32)
    blkn = 1024
    for nb in range(NP // blkn):
        blk = h_ref[pl.ds(nb * blkn, blkn), :]
        bb = bc_ref[pl.ds(nb * blkn, blkn), :]
        bbr = br_ref[:, pl.ds(nb * blkn, blkn)]
        msk = (bb >= 0).astype(jnp.float32)
        lanei = lax.broadcasted_iota(jnp.int32, (blkn, HP), 1)
        blk2 = jnp.where(lanei == HP - 1, msk, blk)
        gi = lax.broadcasted_iota(jnp.int32, (GG, blkn), 0)
        p = (gi == bbr).astype(jnp.float32)
        acc = acc + jnp.dot(p, blk2, preferred_element_type=jnp.float32)
    counts = acc[:, HP - 1:HP]
    hg = acc / jnp.maximum(counts, 1.0)
    z = jnp.maximum(
        jnp.dot(hg, w1_ref[...], preferred_element_type=jnp.float32)
        + b1_ref[...], 0.0)
    o_ref[...] = jnp.dot(
        z, w2_ref[...], preferred_element_type=jnp.float32) + b2_ref[...]


def _tc(body, out_shape, *args):
    return pl.pallas_call(
        body, out_shape=out_shape,
        compiler_params=pltpu.CompilerParams(
            vmem_limit_bytes=120 * 1024 * 1024))(*args)


def kernel(x, edge_index, batch, emb, W, b, gamma, beta, W1, b1, W2, b2):
    f32 = jnp.float32
    i32 = jnp.int32
    x = x.astype(i32)
    edge_index = edge_index.astype(i32)
    batch = batch.astype(i32)

    embp = jnp.zeros((CP, HP), f32).at[:CC, :HH].set(emb)
    Wp = jnp.zeros((LL, HP, HP), f32).at[:, :HH, :HH].set(W)
    bp = jnp.zeros((LL, 1, HP), f32).at[:, 0, :HH].set(b)
    gp = jnp.zeros((LL, 1, HP), f32).at[:, 0, :HH].set(gamma)
    btp = jnp.zeros((LL, 1, HP), f32).at[:, 0, :HH].set(beta)
    W1p = jnp.zeros((HP, HHP), f32).at[:HH, :H2].set(W1)
    b1p = jnp.zeros((1, HHP), f32).at[0, :H2].set(b1)
    W2p = jnp.zeros((HHP, 128), f32).at[:H2, 0].set(W2[:, 0])
    b2p = jnp.zeros((1, 128), f32).at[0, 0].set(b2[0])
    xp = jnp.full((NP, 1), -1, i32).at[:NN].set(x)
    batc = jnp.full((NP, 1), -1, i32).at[:NN, 0].set(batch)
    batr = batc.reshape(1, NP)

    prep_k, conv_k = _sc_kernels()
    rows, rptr, cnts, deg = prep_k(edge_index[0], edge_index[1])

    h, mp, dinv = _tc(
        _embed_body,
        (jax.ShapeDtypeStruct((NP, HP), f32),
         jax.ShapeDtypeStruct((NP, HP), f32),
         jax.ShapeDtypeStruct((NP, 1), f32)),
        xp, embp, Wp[0], deg.reshape(NP, 1))

    for i in range(LL):
        s = conv_k(mp, rows, rptr, cnts).reshape(NP, HP)
        hc, st = _tc(
            _fuse_a_body,
            (jax.ShapeDtypeStruct((NP, HP), f32),
             jax.ShapeDtypeStruct((2, HP), f32)),
            s, mp, dinv, bp[i])
        if i < LL - 1:
            h, mp = _tc(
                _make_fuse_b_body(False),
                (jax.ShapeDtypeStruct((NP, HP), f32),
                 jax.ShapeDtypeStruct((NP, HP), f32)),
                hc, st, h, dinv, gp[i], btp[i], Wp[i + 1])
        else:
            h = _tc(
                _make_fuse_b_body(True),
                jax.ShapeDtypeStruct((NP, HP), f32),
                hc, st, h, dinv, gp[i], btp[i])

    o = _tc(
        _pool_body,
        jax.ShapeDtypeStruct((GG, 128), f32),
        h, batc, batr, W1p, b1p, W2p, b2p)
    return o[:, :1]


# speedup vs baseline: 1.8435x; 1.1186x over previous
import functools

import jax
import jax.numpy as jnp
from jax import lax
from jax.experimental import pallas as pl
from jax.experimental.pallas import tpu as pltpu
from jax.experimental.pallas import tpu_sc as plsc

NN = 10000
EE = 320000
HH = 145
LL = 4
CC = 21
GG = 512
H2 = 72

NC, NS, LANE = 2, 16, 16
NWK = NC * NS
R = 320
NP = NWK * R
HP = 160
CP = 32
HHP = 128
CAP = 16384
WE = 2000
NWIN = EE // WE
K = 128
RP = 336
NVEC = HP // LANE

def _wid():
    return lax.axis_index("s") * NC + lax.axis_index("c")


@functools.cache
def _sc_kernels():
    mesh = plsc.VectorSubcoreMesh(
        core_axis_name="c", subcore_axis_name="s",
        num_cores=NC, num_subcores=NS)
    cparams = pltpu.CompilerParams(needs_layout_passes=False,
                                   use_tc_tiling_on_sc=False)
    prep = pl.kernel(
        _prep_body,
        compiler_params=cparams,
        out_type=[
            jax.ShapeDtypeStruct((NWK, CAP), jnp.int32),
            jax.ShapeDtypeStruct((NWK, RP), jnp.int32),
            jax.ShapeDtypeStruct((NWK, 16), jnp.int32),
            jax.ShapeDtypeStruct((NP,), jnp.int32),
        ],
        mesh=mesh,
        scratch_types=[
            pltpu.VMEM((WE,), jnp.int32),
            pltpu.VMEM((WE,), jnp.int32),
            pltpu.VMEM((WE,), jnp.int32),
            pltpu.VMEM((WE,), jnp.int32),
            pltpu.VMEM((CAP,), jnp.int32),
            pltpu.VMEM((CAP,), jnp.int32),
            pltpu.VMEM((CAP,), jnp.int32),
            pltpu.VMEM((RP * 16,), jnp.int32),
            pltpu.VMEM((RP,), jnp.int32),
            pltpu.VMEM((RP,), jnp.int32),
            pltpu.VMEM((R,), jnp.int32),
            pltpu.VMEM((16,), jnp.int32),
            pltpu.SemaphoreType.DMA,
            pltpu.SemaphoreType.DMA,
            pltpu.SemaphoreType.DMA,
            pltpu.SemaphoreType.DMA,
        ],
    )
    conv = pl.kernel(
        _conv_body,
        compiler_params=cparams,
        out_type=jax.ShapeDtypeStruct((NP * HP,), jnp.float32),
        mesh=mesh,
        scratch_types=[
            pltpu.VMEM(((R + 1) * HP,), jnp.float32),
            pltpu.VMEM((K,), jnp.int32),
            pltpu.VMEM((K,), jnp.int32),
            pltpu.VMEM((K, HP), jnp.float32),
            pltpu.VMEM((K, HP), jnp.float32),
            pltpu.VMEM((RP,), jnp.int32),
            pltpu.VMEM((16,), jnp.int32),
            pltpu.SemaphoreType.DMA,
            pltpu.SemaphoreType.DMA,
            pltpu.SemaphoreType.DMA,
            pltpu.SemaphoreType.DMA,
        ],
    )
    return prep, conv


def _prep_body(erow, ecol, rows_o, rp_o, cnts_o, deg_o,
               colw0, roww0, colw1, roww1, lrows, lcols, srows,
               degacc, rp, cursor, degcomp, cntv, cs0, rs0, cs1, rs1):
    wid = _wid()
    lo = wid * R

    def issue(slot_col, slot_row, csem, rsem, w):
        pltpu.async_copy(ecol.at[pl.ds(w * WE, WE)], slot_col, csem)
        pltpu.async_copy(erow.at[pl.ds(w * WE, WE)], slot_row, rsem)

    def wait(slot_col, slot_row, csem, rsem):
        pltpu.make_async_copy(ecol.at[pl.ds(0, WE)], slot_col, csem).wait()
        pltpu.make_async_copy(erow.at[pl.ds(0, WE)], slot_row, rsem).wait()

    issue(colw0, roww0, cs0, rs0, 0)
    issue(colw1, roww1, cs1, rs1, 1)

    def process(colw, roww, cur):
        @plsc.parallel_loop(0, WE // LANE, carry=cur)
        def group(g, cur):
            col = colw[pl.ds(g * LANE, LANE)]
            row = roww[pl.ds(g * LANE, LANE)]
            m = (col >= lo) & (col < lo + R)
            safe = jnp.minimum(cur, CAP - 160)
            mi = jnp.where(m, jnp.int32(1), jnp.int32(0))
            pos = safe + plsc.cumsum(mi) - 1
            plsc.store_scatter(lcols, [pos], col - lo, mask=m)
            plsc.store_scatter(lrows, [pos], row, mask=m)
            cnt = jnp.sum(mi)
            return jnp.minimum(cur + cnt, CAP - 160)
        return group

    def pair(t, cur):
        wait(colw0, roww0, cs0, rs0)
        cur = process(colw0, roww0, cur)

        @pl.when(2 * t + 2 < NWIN)
        def _():
            issue(colw0, roww0, cs0, rs0, 2 * t + 2)

        wait(colw1, roww1, cs1, rs1)
        cur = process(colw1, roww1, cur)

        @pl.when(2 * t + 3 < NWIN)
        def _():
            issue(colw1, roww1, cs1, rs1, 2 * t + 3)
        return cur

    cur = lax.fori_loop(0, NWIN // 2, pair, jnp.int32(0))

    base = jnp.minimum(cur, CAP - 160)
    for g in range(K // LANE):
        lcols[pl.ds(base + g * LANE, LANE)] = jnp.full((LANE,), R, jnp.int32)
        lrows[pl.ds(base + g * LANE, LANE)] = jnp.zeros((LANE,), jnp.int32)
    cnt_final = ((base + K - 1) // K) * K

    cntv[pl.ds(0, LANE)] = jnp.full((LANE,), cnt_final, jnp.int32)
    pltpu.sync_copy(cntv, cnts_o.at[wid])

    iota = lax.broadcasted_iota(jnp.int32, (LANE,), 0)

    def zero_deg(i, _):
        degacc[pl.ds(i * LANE, LANE)] = jnp.zeros((LANE,), jnp.int32)
        return 0
    lax.fori_loop(0, RP, zero_deg, 0)

    def hist_group2(g, _):
        cv = lcols[pl.ds(g * LANE, LANE)]
        cs, _unused = plsc.sort_key_val(cv, cv)
        cprev = jnp.take(cs, jnp.maximum(iota - 1, 0))
        same = (iota > 0) & (cs == cprev)
        brk = jnp.where(same, jnp.int32(0), iota)
        rank = iota - plsc.cummax(brk)
        cnext = jnp.take(cs, jnp.minimum(iota + 1, LANE - 1))
        last = (iota == LANE - 1) | (cs != cnext)
        plsc.addupdate_scatter(degacc, [cs * LANE], rank + 1, mask=last)
        return 0
    lax.fori_loop(0, cnt_final // LANE, hist_group2, 0)

    def compact(g, _):
        idx = (g * LANE + iota) * LANE
        degcomp[pl.ds(g * LANE, LANE)] = plsc.load_gather(degacc, [idx])
        return 0
    lax.fori_loop(0, R // LANE, compact, 0)
    pltpu.sync_copy(degcomp, deg_o.at[pl.ds(wid * R, R)])

    def prefix(g, tot):
        v = plsc.load_gather(degacc, [(g * LANE + iota) * LANE])
        ex = plsc.cumsum(v) - v
        rp[pl.ds(g * LANE, LANE)] = ex + tot
        cursor[pl.ds(g * LANE, LANE)] = ex + tot
        return tot + jnp.sum(v)
    lax.fori_loop(0, RP // LANE, prefix, jnp.int32(0))
    pltpu.sync_copy(rp, rp_o.at[wid])

    def permute(g, _):
        cv = lcols[pl.ds(g * LANE, LANE)]
        rv = lrows[pl.ds(g * LANE, LANE)]
        cs, rs = plsc.sort_key_val(cv, rv)
        cprev = jnp.take(cs, jnp.maximum(iota - 1, 0))
        same = (iota > 0) & (cs == cprev)
        brk = jnp.where(same, jnp.int32(0), iota)
        rank = iota - plsc.cummax(brk)
        cnext = jnp.take(cs, jnp.minimum(iota + 1, LANE - 1))
        last = (iota == LANE - 1) | (cs != cnext)
        pos = plsc.load_gather(cursor, [cs]) + rank
        plsc.store_scatter(cursor, [cs], pos + 1, mask=last)
        plsc.store_scatter(srows, [pos], rs)
        return 0
    lax.fori_loop(0, cnt_final // LANE, permute, 0)
    pltpu.sync_copy(srows, rows_o.at[wid])


def _conv_body(mp, rows, rp_h, cnts, out,
               acc, idx0, idx1, g0, g1, rpv, cntv,
               is0, is1, gs0, gs1):
    wid = _wid()
    iota = lax.broadcasted_iota(jnp.int32, (LANE,), 0)
    pltpu.sync_copy(cnts.at[wid], cntv)
    cv0 = cntv[pl.ds(0, LANE)]
    nw = jnp.sum(jnp.where(iota == 0, cv0, jnp.int32(0))) // K
    pltpu.sync_copy(rp_h.at[wid], rpv)

    @plsc.parallel_loop(0, (R + 1) * HP // LANE, unroll=4)
    def zero(i):
        acc[pl.ds(i * LANE, LANE)] = jnp.zeros((LANE,), jnp.float32)

    def rp_at(c):
        v = rpv[pl.ds((c // LANE) * LANE, LANE)]
        return jnp.sum(jnp.where(iota == (c % LANE), v, jnp.int32(0)))

    def accum(g, w, c0):
        e0 = w * K
        e1 = e0 + K

        def row_cond(st):
            c, rpc = st
            return (c < R) & (rpc < e1)

        def row_body(st):
            c, rpc = st
            rpn = rp_at(c + 1)
            s = jnp.maximum(rpc, e0)
            t = jnp.minimum(rpn, e1)

            def edge2(i, regs):
                je = s - e0 + 2 * i
                return tuple(
                    regs[k] + g[je, pl.ds(k * LANE, LANE)]
                    + g[je + 1, pl.ds(k * LANE, LANE)]
                    for k in range(NVEC))
            n = t - s
            regs = lax.fori_loop(
                0, n // 2, edge2,
                tuple(jnp.zeros((LANE,), jnp.float32) for _ in range(NVEC)))

            def tail(regs):
                je = t - 1 - e0
                return tuple(
                    regs[k] + g[je, pl.ds(k * LANE, LANE)]
                    for k in range(NVEC))
            regs = lax.cond(n % 2 == 1, tail, lambda r: r, regs)
            base = c * HP
            for k in range(NVEC):
                plsc.addupdate(acc.at[pl.ds(base + k * LANE, LANE)], regs[k])
            return (c + 1, rpn)

        c_ex, rp_ex = lax.while_loop(row_cond, row_body, (c0, rp_at(c0)))
        return jnp.where(rp_ex > e1, c_ex - 1, c_ex)

    @pl.when(nw > 0)
    def _():
        pltpu.async_copy(rows.at[wid, pl.ds(0, K)], idx0, is0)

    @pl.when(nw > 1)
    def _():
        pltpu.async_copy(rows.at[wid, pl.ds(K, K)], idx1, is1)

    @pl.when(nw > 0)
    def _():
        pltpu.make_async_copy(rows.at[wid, pl.ds(0, K)], idx0, is0).wait()
        pltpu.async_copy(mp.at[idx0], g0, gs0)

    def pair(t, c0):
        w0 = 2 * t
        w1 = 2 * t + 1
        pltpu.make_async_copy(mp.at[idx0], g0, gs0).wait()

        @pl.when(w0 + 2 < nw)
        def _():
            pltpu.async_copy(rows.at[wid, pl.ds((w0 + 2) * K, K)], idx0, is0)

        @pl.when(w1 < nw)
        def _():
            pltpu.make_async_copy(rows.at[wid, pl.ds(0, K)], idx1, is1).wait()
            pltpu.async_copy(mp.at[idx1], g1, gs1)

        c0 = accum(g0, w0, c0)

        def snd():
            pltpu.make_async_copy(mp.at[idx1], g1, gs1).wait()

            @pl.when(w1 + 2 < nw)
            def _():
                pltpu.async_copy(rows.at[wid, pl.ds((w1 + 2) * K, K)], idx1, is1)

            @pl.when(w0 + 2 < nw)
            def _():
                pltpu.make_async_copy(rows.at[wid, pl.ds(0, K)], idx0, is0).wait()
                pltpu.async_copy(mp.at[idx0], g0, gs0)

            return accum(g1, w1, c0)

        return lax.cond(w1 < nw, snd, lambda: c0)

    lax.fori_loop(0, (nw + 1) // 2, pair, jnp.int32(0))
    pltpu.sync_copy(acc.at[pl.ds(0, R * HP)], out.at[pl.ds(wid * R * HP, R * HP)])


def _embed_body(x_ref, emb_ref, w0_ref, deg_ref, h_ref, mp_ref, dinv_ref):
    xv = x_ref[...]
    ohi = lax.broadcasted_iota(jnp.int32, (NP, CP), 1)
    oh = (ohi == xv).astype(jnp.float32)
    h0 = jnp.dot(oh, emb_ref[...], preferred_element_type=jnp.float32)
    rmask = (lax.broadcasted_iota(jnp.int32, (NP, 1), 0) < NN).astype(jnp.float32)
    degf = deg_ref[...].astype(jnp.float32) + rmask
    dinv = rmask * lax.rsqrt(jnp.maximum(degf, 1.0))
    m0 = jnp.dot(h0, w0_ref[...], preferred_element_type=jnp.float32)
    h_ref[...] = h0
    mp_ref[...] = dinv * m0
    dinv_ref[...] = dinv


def _fuse_a_body(s_ref, mp_ref, dinv_ref, b_ref, hc_ref, st_ref):
    hc = dinv_ref[...] * (s_ref[...] + mp_ref[...]) + b_ref[...]
    s1 = jnp.sum(hc, axis=0, keepdims=True)
    mean = s1 * (1.0 / NN)
    rmask = (lax.broadcasted_iota(jnp.int32, (NP, 1), 0) < NN).astype(jnp.float32)
    dcen = (hc - mean) * rmask
    s2 = jnp.sum(dcen * dcen, axis=0, keepdims=True)
    var = s2 * (1.0 / NN)
    hc_ref[...] = hc
    st_ref[...] = jnp.concatenate([mean, lax.rsqrt(var + 1e-5)], axis=0)


def _make_fuse_b_body(last):
    def body(*refs):
        if last:
            (hc_ref, st_ref, h_ref, dinv_ref, g_ref, bt_ref, ho_ref) = refs
        else:
            (hc_ref, st_ref, h_ref, dinv_ref, g_ref, bt_ref, wn_ref,
             ho_ref, mpo_ref) = refs
        mean = st_ref[0:1, :]
        rstd = st_ref[1:2, :]
        hn = (hc_ref[...] - mean) * rstd * g_ref[...] + bt_ref[...]
        rmask = (lax.broadcasted_iota(jnp.int32, (NP, 1), 0) < NN).astype(jnp.float32)
        hr = jnp.maximum(hn, 0.0) * rmask + h_ref[...]
        ho_ref[...] = hr
        if not last:
            mpo_ref[...] = dinv_ref[...] * jnp.dot(
                hr, wn_ref[...], preferred_element_type=jnp.float32)
    return body


def _pool_body(h_ref, bc_ref, br_ref, w1_ref, b1_ref, w2_ref, b2_ref, o_ref):
    acc = jnp.zeros((GG, HP), jnp.float32)
    blkn = 1024
    for nb in range(NP // blkn):
        blk = h_ref[pl.ds(nb * blkn, blkn), :]
        bb = bc_ref[pl.ds(nb * blkn, blkn), :]
        bbr = br_ref[:, pl.ds(nb * blkn, blkn)]
        msk = (bb >= 0).astype(jnp.float32)
        lanei = lax.broadcasted_iota(jnp.int32, (blkn, HP), 1)
        blk2 = jnp.where(lanei == HP - 1, msk, blk)
        gi = lax.broadcasted_iota(jnp.int32, (GG, blkn), 0)
        p = (gi == bbr).astype(jnp.float32)
        acc = acc + jnp.dot(p, blk2, preferred_element_type=jnp.float32)
    counts = acc[:, HP - 1:HP]
    hg = acc / jnp.maximum(counts, 1.0)
    z = jnp.maximum(
        jnp.dot(hg, w1_ref[...], preferred_element_type=jnp.float32)
        + b1_ref[...], 0.0)
    o_ref[...] = jnp.dot(
        z, w2_ref[...], preferred_element_type=jnp.float32) + b2_ref[...]


def _tc(body, out_shape, *args):
    return pl.pallas_call(
        body, out_shape=out_shape,
        compiler_params=pltpu.CompilerParams(
            vmem_limit_bytes=120 * 1024 * 1024))(*args)


def kernel(x, edge_index, batch, emb, W, b, gamma, beta, W1, b1, W2, b2):
    f32 = jnp.float32
    i32 = jnp.int32
    x = x.astype(i32)
    edge_index = edge_index.astype(i32)
    batch = batch.astype(i32)

    embp = jnp.zeros((CP, HP), f32).at[:CC, :HH].set(emb)
    Wp = jnp.zeros((LL, HP, HP), f32).at[:, :HH, :HH].set(W)
    bp = jnp.zeros((LL, 1, HP), f32).at[:, 0, :HH].set(b)
    gp = jnp.zeros((LL, 1, HP), f32).at[:, 0, :HH].set(gamma)
    btp = jnp.zeros((LL, 1, HP), f32).at[:, 0, :HH].set(beta)
    W1p = jnp.zeros((HP, HHP), f32).at[:HH, :H2].set(W1)
    b1p = jnp.zeros((1, HHP), f32).at[0, :H2].set(b1)
    W2p = jnp.zeros((HHP, 128), f32).at[:H2, 0].set(W2[:, 0])
    b2p = jnp.zeros((1, 128), f32).at[0, 0].set(b2[0])
    xp = jnp.full((NP, 1), -1, i32).at[:NN].set(x)
    batc = jnp.full((NP, 1), -1, i32).at[:NN, 0].set(batch)
    batr = batc.reshape(1, NP)

    prep_k, conv_k = _sc_kernels()
    rows, rptr, cnts, deg = prep_k(edge_index[0], edge_index[1])

    h, mp, dinv = _tc(
        _embed_body,
        (jax.ShapeDtypeStruct((NP, HP), f32),
         jax.ShapeDtypeStruct((NP, HP), f32),
         jax.ShapeDtypeStruct((NP, 1), f32)),
        xp, embp, Wp[0], deg.reshape(NP, 1))

    for i in range(LL):
        s = conv_k(mp, rows, rptr, cnts).reshape(NP, HP)
        hc, st = _tc(
            _fuse_a_body,
            (jax.ShapeDtypeStruct((NP, HP), f32),
             jax.ShapeDtypeStruct((2, HP), f32)),
            s, mp, dinv, bp[i])
        if i < LL - 1:
            h, mp = _tc(
                _make_fuse_b_body(False),
                (jax.ShapeDtypeStruct((NP, HP), f32),
                 jax.ShapeDtypeStruct((NP, HP), f32)),
                hc, st, h, dinv, gp[i], btp[i], Wp[i + 1])
        else:
            h = _tc(
                _make_fuse_b_body(True),
                jax.ShapeDtypeStruct((NP, HP), f32),
                hc, st, h, dinv, gp[i], btp[i])

    o = _tc(
        _pool_body,
        jax.ShapeDtypeStruct((GG, 128), f32),
        h, batc, batr, W1p, b1p, W2p, b2p)
    return o[:, :1]
